# baseline (speedup 1.0000x reference)
import functools

import jax
import jax.numpy as jnp
from jax import lax
from jax.experimental import pallas as pl
from jax.experimental.pallas import tpu as pltpu
from jax.experimental.pallas import tpu_sc as plsc

N_PEAKS = 12000
BLUR_K = 5
BLUR_SIGMA = 1.5

_B, _F, _T = 16, 256, 1024
_ITEM = _F * _T
_QITEM = _ITEM // 4
_NITEMS = 8
_CH = 2048
_NCHUNKS = _QITEM // _CH
_NBUF = 12032
_OUT_ITEM = 3 * N_PEAKS
_OUT_LEN = _NITEMS * _OUT_ITEM
_WINDOW = 8


def _dense_body(kb_ref, x_ref, o_ref):
    x = x_ref[0]
    F, T = x.shape
    ninf = jnp.float32(-jnp.inf)
    fidx = lax.broadcasted_iota(jnp.int32, (F, T), 0)
    tidx = lax.broadcasted_iota(jnp.int32, (F, T), 1)

    def peaks(f):
        lt = jnp.where(tidx == T - 1, ninf, jnp.roll(f, -1, axis=1))
        rt = jnp.where(tidx == 0, ninf, jnp.roll(f, 1, axis=1))
        mt = jnp.maximum(f, jnp.maximum(lt, rt))
        lf = jnp.where(fidx == F - 1, ninf, jnp.roll(f, -1, axis=0))
        rf = jnp.where(fidx == 0, ninf, jnp.roll(f, 1, axis=0))
        mf = jnp.maximum(f, jnp.maximum(lf, rf))
        isp = (f == mt) & (f == mf)
        mn = jnp.min(f)
        mx = jnp.max(f)
        fn = (f - mn) / (mx - mn)
        return jnp.where(isp, fn, jnp.float32(0.0))

    def conv_axis(f, axis, n):
        idx = fidx if axis == 0 else tidx
        fb = f.astype(jnp.bfloat16).astype(jnp.float32)
        r1 = jnp.roll(fb, -1, axis)
        rm1 = jnp.roll(fb, 1, axis)
        r2 = jnp.roll(fb, -2, axis)
        rm2 = jnp.roll(fb, 2, axis)
        sm2 = jnp.where(idx == 1, fb, jnp.where(idx == 0, r2, rm2))
        sm1 = jnp.where(idx == 0, r1, rm1)
        sp1 = jnp.where(idx == n - 1, rm1, r1)
        sp2 = jnp.where(idx == n - 2, fb, jnp.where(idx == n - 1, rm2, r2))
        kb = [kb_ref[d].astype(jnp.bfloat16).astype(jnp.float32)
              for d in range(5)]
        acc = kb[0] * sm2
        acc = acc + kb[1] * sm1
        acc = acc + kb[2] * fb
        acc = acc + kb[3] * sp1
        acc = acc + kb[4] * sp2
        return acc

    p1 = peaks(x)
    y = conv_axis(p1, 0, F)
    feat = conv_axis(y, 1, T)
    o_ref[0] = peaks(feat)


def _peak_map(spec_tensor):
    B, F, T = spec_tensor.shape
    half = (BLUR_K - 1) * 0.5
    t = jnp.linspace(-half, half, BLUR_K)
    pdf = jnp.exp(-0.5 * (t / BLUR_SIGMA) ** 2)
    k1 = (pdf / pdf.sum()).astype(jnp.float32)

    return pl.pallas_call(
        _dense_body,
        grid=(B,),
        in_specs=[
            pl.BlockSpec(memory_space=pltpu.SMEM),
            pl.BlockSpec((1, F, T), lambda i: (i, 0, 0)),
        ],
        out_specs=pl.BlockSpec((1, F, T), lambda i: (i, 0, 0)),
        out_shape=jax.ShapeDtypeStruct((B, F, T), jnp.float32),
    )(k1, spec_tensor)


def _sc_body(p_ref, o_ref, inbuf0, inbuf1, valbuf, idxbuf, pval1, pidx1,
             pval2, pidx2, pval3, pidx3, vals, vtail, zbuf, cntv, cnt3,
             sval1, sidx1, sval2, sidx2, sval3, sidx3, cnt_smem,
             sem_a, sem_b, sem_sc):
    c = lax.axis_index("c")
    s = lax.axis_index("s")
    il = s // 4
    q = s % 4
    item = c * 4 + il
    iota16 = lax.iota(jnp.int32, 16)

    cnt_smem[0] = 0
    cnt_smem[1] = 0
    cnt_smem[2] = 0

    def zfill(t, carry):
        zbuf[pl.ds(t * 16, 16)] = jnp.zeros((16,), jnp.float32)
        return carry
    lax.fori_loop(0, 150, zfill, 0)
    zstart = item * _OUT_ITEM + q * 9600
    for j in range(4):
        @pl.when((q < 3) | (j < 3))
        def _(j=j):
            pltpu.sync_copy(zbuf, o_ref.at[pl.ds(zstart + j * 2400, 2400)])

    src_base = item * _ITEM + q * _QITEM
    lin_base = q * _QITEM
    sems = (sem_a, sem_b)
    bufs = (inbuf0, inbuf1)

    def start_copy(b, ch):
        pltpu.async_copy(p_ref.at[pl.ds(src_base + ch * _CH, _CH)],
                         bufs[b], sems[b])

    def wait_copy(b, ch):
        pltpu.make_async_copy(p_ref.at[pl.ds(src_base + ch * _CH, _CH)],
                              bufs[b], sems[b]).wait()

    def process(buf, cb, pos):
        @plsc.parallel_loop(0, _CH // 16, carry=pos, unroll=4)
        def step(t, pos):
            v = buf[pl.ds(t * 16, 16)]
            m = v != 0.0
            mi = jnp.where(m, 1, 0).astype(jnp.int32)
            rank = pos + plsc.cumsum(mi) - 1
            lin = cb + t * 16 + iota16
            mstore = m & (rank < N_PEAKS)
            plsc.store_scatter(valbuf, [rank], v, mask=mstore)
            plsc.store_scatter(idxbuf, [rank], lin, mask=mstore)
            return pos + plsc.all_reduce_population_count(m)

        return step

    start_copy(0, 0)

    def pair_body(g, pos):
        for b in range(2):
            ch = g * 2 + b

            @pl.when(ch + 1 < _NCHUNKS)
            def _(b=b, ch=ch):
                start_copy(1 - b, ch + 1)

            wait_copy(b, ch)
            pos = process(bufs[b], lin_base + ch * _CH, pos)
        return pos

    pos = lax.fori_loop(0, _NCHUNKS // 2, pair_body,
                        jnp.zeros((16,), jnp.int32))

    plsc.subcore_barrier()

    @pl.when(q > 0)
    def _():
        plsc.fetch_and_add(cnt_smem.at[q - 1], pos[0], subcore_id=s - q)

    @pl.when(q == 1)
    def _():
        pltpu.sync_copy(valbuf, sval1.at[il])
        pltpu.sync_copy(idxbuf, sidx1.at[il])

    @pl.when(q == 2)
    def _():
        pltpu.sync_copy(valbuf, sval2.at[il])
        pltpu.sync_copy(idxbuf, sidx2.at[il])

    @pl.when(q == 3)
    def _():
        pltpu.sync_copy(valbuf, sval3.at[il])
        pltpu.sync_copy(idxbuf, sidx3.at[il])

    plsc.subcore_barrier()

    @pl.when(q == 0)
    def _phase_b():
        pltpu.sync_copy(sval1.at[il], pval1)
        pltpu.sync_copy(sidx1.at[il], pidx1)
        pltpu.sync_copy(sval2.at[il], pval2)
        pltpu.sync_copy(sidx2.at[il], pidx2)
        pltpu.sync_copy(sval3.at[il], pval3)
        pltpu.sync_copy(sidx3.at[il], pidx3)
        c0 = pos[0]
        b1 = c0 + cnt_smem[0]
        b2 = b1 + cnt_smem[1]
        n_tot = b2 + cnt_smem[2]
        n_keep = jnp.minimum(n_tot, N_PEAKS)
        nfull = jnp.minimum((n_keep + 127) // 128, 93)
        ngrp = (nfull + _WINDOW - 1) // _WINDOW
        base_out = item * _OUT_ITEM

        def build16(eb, store):
            ev = eb + iota16
            s0 = ev < c0
            s1 = ev < b1
            s2 = ev < b2
            i0 = jnp.minimum(ev, _NBUF - 1)
            i1 = jnp.clip(ev - c0, 0, _NBUF - 1)
            i2 = jnp.clip(ev - b1, 0, _NBUF - 1)
            i3 = jnp.clip(ev - b2, 0, _NBUF - 1)
            l = jnp.where(
                s0, plsc.load_gather(idxbuf, [i0]),
                jnp.where(s1, plsc.load_gather(pidx1, [i1]),
                          jnp.where(s2, plsc.load_gather(pidx2, [i2]),
                                    plsc.load_gather(pidx3, [i3]))))
            v = jnp.where(
                s0, plsc.load_gather(valbuf, [i0]),
                jnp.where(s1, plsc.load_gather(pval1, [i1]),
                          jnp.where(s2, plsc.load_gather(pval2, [i2]),
                                    plsc.load_gather(pval3, [i3]))))
            valid = ev < n_keep
            z = jnp.float32(0.0)
            fi = (l >> 10).astype(jnp.float32) * (1.0 / _F)
            ti = (l & (_T - 1)).astype(jnp.float32) * (1.0 / _T)
            store(0, jnp.where(valid, fi, z))
            store(1, jnp.where(valid, ti, z))
            store(2, jnp.where(valid, v, z))

        def fire(b, ch2):
            for r in range(3):
                pltpu.async_copy(
                    vals.at[r, b],
                    o_ref.at[pl.ds(base_out + r * N_PEAKS + ch2 * 128, 128)],
                    sem_sc)

        def drain(b, ch2):
            for r in range(3):
                pltpu.make_async_copy(
                    vals.at[r, b],
                    o_ref.at[pl.ds(base_out + r * N_PEAKS + ch2 * 128, 128)],
                    sem_sc).wait()

        def group_body(g, carry):
            for b in range(_WINDOW):
                ch2 = g * _WINDOW + b

                @pl.when(ch2 < nfull)
                def _(b=b, ch2=ch2):
                    @pl.when(g > 0)
                    def _():
                        drain(b, ch2 - _WINDOW)

                    for e16 in range(8):
                        def store(r, x, b=b, e16=e16):
                            vals[r, b, pl.ds(e16 * 16, 16)] = x
                        build16(ch2 * 128 + e16 * 16, store)
                    fire(b, ch2)

            return carry

        lax.fori_loop(0, ngrp, group_body, 0)

        @pl.when(n_keep > 11904)
        def _():
            for e16 in range(6):
                def store(r, x, e16=e16):
                    vtail[r, pl.ds(e16 * 16, 16)] = x
                build16(11904 + e16 * 16, store)
            for r in range(3):
                pltpu.async_copy(
                    vtail.at[r],
                    o_ref.at[pl.ds(base_out + r * N_PEAKS + 11904, 96)],
                    sem_sc)
            for r in range(3):
                pltpu.make_async_copy(
                    vtail.at[r],
                    o_ref.at[pl.ds(base_out + r * N_PEAKS + 11904, 96)],
                    sem_sc).wait()

        for b in range(_WINDOW):
            @pl.when(b < nfull)
            def _(b=b):
                last_ch = b + _WINDOW * ((nfull - 1 - b) // _WINDOW)
                drain(b, last_ch)


def _compact(p_flat):
    mesh = plsc.VectorSubcoreMesh(core_axis_name="c", subcore_axis_name="s")
    f = functools.partial(
        pl.kernel,
        out_type=jax.ShapeDtypeStruct((_OUT_LEN,), jnp.float32),
        name="compact8",
        mesh=mesh,
        scratch_types=[
            pltpu.VMEM((_CH,), jnp.float32),
            pltpu.VMEM((_CH,), jnp.float32),
            pltpu.VMEM((_NBUF,), jnp.float32),
            pltpu.VMEM((_NBUF,), jnp.int32),
            pltpu.VMEM((_NBUF,), jnp.float32),
            pltpu.VMEM((_NBUF,), jnp.int32),
            pltpu.VMEM((_NBUF,), jnp.float32),
            pltpu.VMEM((_NBUF,), jnp.int32),
            pltpu.VMEM((_NBUF,), jnp.float32),
            pltpu.VMEM((_NBUF,), jnp.int32),
            pltpu.VMEM((3, _WINDOW, 128), jnp.float32),
            pltpu.VMEM((3, 96), jnp.float32),
            pltpu.VMEM((2400,), jnp.float32),
            pltpu.VMEM((16,), jnp.int32),
            pltpu.VMEM((3, 16), jnp.int32),
            pltpu.VMEM_SHARED((4, _NBUF), jnp.float32),
            pltpu.VMEM_SHARED((4, _NBUF), jnp.int32),
            pltpu.VMEM_SHARED((4, _NBUF), jnp.float32),
            pltpu.VMEM_SHARED((4, _NBUF), jnp.int32),
            pltpu.VMEM_SHARED((4, _NBUF), jnp.float32),
            pltpu.VMEM_SHARED((4, _NBUF), jnp.int32),
            pltpu.SMEM((3,), jnp.int32),
            pltpu.SemaphoreType.DMA,
            pltpu.SemaphoreType.DMA,
            pltpu.SemaphoreType.DMA,
        ],
        compiler_params=pltpu.CompilerParams(needs_layout_passes=False),
    )(_sc_body)
    return f(p_flat)


def kernel(spec_tensor):
    B, F, T = spec_tensor.shape
    pad = jnp.zeros((64,), jnp.float32)
    pa = _peak_map(spec_tensor[:_NITEMS])
    oa = _compact(jnp.concatenate([jnp.reshape(pa, (-1,)), pad]))
    pb = _peak_map(spec_tensor[_NITEMS:])
    ob = _compact(jnp.concatenate([jnp.reshape(pb, (-1,)), pad]))
    out = jnp.concatenate([oa, ob])
    return jnp.reshape(out, (B, 3, N_PEAKS))


# speedup vs baseline: 1.3424x; 1.3424x over previous
import functools

import jax
import jax.numpy as jnp
from jax import lax
from jax.experimental import pallas as pl
from jax.experimental.pallas import tpu as pltpu
from jax.experimental.pallas import tpu_sc as plsc

N_PEAKS = 12000
BLUR_K = 5
BLUR_SIGMA = 1.5

_B, _F, _T = 16, 256, 1024
_ITEM = _F * _T
_HALF = _ITEM // 2
_CH = 8192
_NCHUNKS = _HALF // _CH
_NBUF = 12032
_NCH_OUT = 94
_OUT_ITEM = 3 * N_PEAKS
_OUT_LEN = _B * _OUT_ITEM
_WINDOW = 8


def _dense_body(kb_ref, x_ref, o_ref):
    x = x_ref[0]
    F, T = x.shape
    ninf = jnp.float32(-jnp.inf)
    fidx = lax.broadcasted_iota(jnp.int32, (F, T), 0)
    tidx = lax.broadcasted_iota(jnp.int32, (F, T), 1)

    def peaks(f):
        lt = jnp.where(tidx == T - 1, ninf, jnp.roll(f, -1, axis=1))
        rt = jnp.where(tidx == 0, ninf, jnp.roll(f, 1, axis=1))
        mt = jnp.maximum(f, jnp.maximum(lt, rt))
        lf = jnp.where(fidx == F - 1, ninf, jnp.roll(f, -1, axis=0))
        rf = jnp.where(fidx == 0, ninf, jnp.roll(f, 1, axis=0))
        mf = jnp.maximum(f, jnp.maximum(lf, rf))
        isp = (f == mt) & (f == mf)
        mn = jnp.min(f)
        mx = jnp.max(f)
        fn = (f - mn) / (mx - mn)
        return jnp.where(isp, fn, jnp.float32(0.0))

    def conv_axis(f, axis, n):
        idx = fidx if axis == 0 else tidx
        fb = f.astype(jnp.bfloat16).astype(jnp.float32)
        r1 = jnp.roll(fb, -1, axis)
        rm1 = jnp.roll(fb, 1, axis)
        r2 = jnp.roll(fb, -2, axis)
        rm2 = jnp.roll(fb, 2, axis)
        sm2 = jnp.where(idx == 1, fb, jnp.where(idx == 0, r2, rm2))
        sm1 = jnp.where(idx == 0, r1, rm1)
        sp1 = jnp.where(idx == n - 1, rm1, r1)
        sp2 = jnp.where(idx == n - 2, fb, jnp.where(idx == n - 1, rm2, r2))
        kb = [kb_ref[d].astype(jnp.bfloat16).astype(jnp.float32)
              for d in range(5)]
        acc = kb[0] * sm2
        acc = acc + kb[1] * sm1
        acc = acc + kb[2] * fb
        acc = acc + kb[3] * sp1
        acc = acc + kb[4] * sp2
        return acc

    p1 = peaks(x)
    y = conv_axis(p1, 0, F)
    feat = conv_axis(y, 1, T)
    o_ref[0] = peaks(feat)


def _peak_map(spec_tensor):
    B, F, T = spec_tensor.shape
    half = (BLUR_K - 1) * 0.5
    t = jnp.linspace(-half, half, BLUR_K)
    pdf = jnp.exp(-0.5 * (t / BLUR_SIGMA) ** 2)
    k1 = (pdf / pdf.sum()).astype(jnp.float32)

    return pl.pallas_call(
        _dense_body,
        grid=(B,),
        in_specs=[
            pl.BlockSpec(memory_space=pltpu.SMEM),
            pl.BlockSpec((1, F, T), lambda i: (i, 0, 0)),
        ],
        out_specs=pl.BlockSpec((1, F, T), lambda i: (i, 0, 0)),
        out_shape=jax.ShapeDtypeStruct((B, F, T), jnp.float32),
    )(k1, spec_tensor)


def _sc_body(p_ref, o_ref, inbuf0, inbuf1, valbuf, idxbuf, pval, pidx, vals,
             vtail, zbuf, sval, sidx, cnt_smem, sem_a, sem_b, sem_sc):
    c = lax.axis_index("c")
    s = lax.axis_index("s")
    item = c * 8 + s // 2
    il = s // 2
    h = s % 2
    iota16 = lax.iota(jnp.int32, 16)

    cnt_smem[0] = 0

    def zfill(t, carry):
        zbuf[pl.ds(t * 16, 16)] = jnp.zeros((16,), jnp.float32)
        return carry
    lax.fori_loop(0, 125, zfill, 0)
    zstart = item * _OUT_ITEM + h * (_OUT_ITEM // 2)
    for j in range(9):
        pltpu.sync_copy(zbuf, o_ref.at[pl.ds(zstart + j * 2000, 2000)])

    plsc.subcore_barrier()

    src_base = item * _ITEM + h * _HALF
    lin_base = h * _HALF
    sems = (sem_a, sem_b)
    bufs = (inbuf0, inbuf1)

    def start_copy(b, ch):
        pltpu.async_copy(p_ref.at[pl.ds(src_base + ch * _CH, _CH)],
                         bufs[b], sems[b])

    def wait_copy(b, ch):
        pltpu.make_async_copy(p_ref.at[pl.ds(src_base + ch * _CH, _CH)],
                              bufs[b], sems[b]).wait()

    def process(buf, cb, pos):
        @plsc.parallel_loop(0, _CH // 16, carry=pos, unroll=4)
        def step(t, pos):
            v = buf[pl.ds(t * 16, 16)]
            m = v != 0.0
            mi = jnp.where(m, 1, 0).astype(jnp.int32)
            rank = pos + plsc.cumsum(mi) - 1
            lin = cb + t * 16 + iota16
            mstore = m & (rank < N_PEAKS)
            plsc.store_scatter(valbuf, [rank], v, mask=mstore)
            plsc.store_scatter(idxbuf, [rank], lin, mask=mstore)
            return pos + plsc.all_reduce_population_count(m)

        return step

    start_copy(0, 0)

    def pair_body(g, pos):
        for b in range(2):
            ch = g * 2 + b

            @pl.when(ch + 1 < _NCHUNKS)
            def _(b=b, ch=ch):
                start_copy(1 - b, ch + 1)

            wait_copy(b, ch)
            pos = process(bufs[b], lin_base + ch * _CH, pos)
        return pos

    pos = lax.fori_loop(0, _NCHUNKS // 2, pair_body,
                        jnp.zeros((16,), jnp.int32))
    c_tot = pos[0]

    @pl.when(h == 1)
    def _():
        plsc.fetch_and_add(cnt_smem.at[0], c_tot, subcore_id=s - 1)
        pltpu.sync_copy(valbuf, sval.at[il])
        pltpu.sync_copy(idxbuf, sidx.at[il])

    plsc.subcore_barrier()

    @pl.when(h == 0)
    def _phase_b():
        pltpu.sync_copy(sval.at[il], pval)
        pltpu.sync_copy(sidx.at[il], pidx)
        c0 = c_tot
        n_keep = jnp.minimum(c0 + cnt_smem[0], N_PEAKS)
        nfull = jnp.minimum((n_keep + 127) // 128, 93)
        ngrp = (nfull + _WINDOW - 1) // _WINDOW
        base_out = item * _OUT_ITEM

        def build16(eb, store):
            ev = eb + iota16
            own = ev < c0
            oi = jnp.minimum(ev, _NBUF - 1)
            pj = jnp.clip(ev - c0, 0, _NBUF - 1)
            lo = plsc.load_gather(idxbuf, [oi])
            lp = plsc.load_gather(pidx, [pj])
            vo = plsc.load_gather(valbuf, [oi])
            vp = plsc.load_gather(pval, [pj])
            l = jnp.where(own, lo, lp)
            v = jnp.where(own, vo, vp)
            valid = ev < n_keep
            z = jnp.float32(0.0)
            fi = (l >> 10).astype(jnp.float32) * (1.0 / _F)
            ti = (l & (_T - 1)).astype(jnp.float32) * (1.0 / _T)
            store(0, jnp.where(valid, fi, z))
            store(1, jnp.where(valid, ti, z))
            store(2, jnp.where(valid, v, z))

        def fire(b, ch2):
            for r in range(3):
                pltpu.async_copy(
                    vals.at[r, b],
                    o_ref.at[pl.ds(base_out + r * N_PEAKS + ch2 * 128, 128)],
                    sem_sc)

        def drain(b, ch2):
            for r in range(3):
                pltpu.make_async_copy(
                    vals.at[r, b],
                    o_ref.at[pl.ds(base_out + r * N_PEAKS + ch2 * 128, 128)],
                    sem_sc).wait()

        def group_body(g, carry):
            for b in range(_WINDOW):
                ch2 = g * _WINDOW + b

                @pl.when(ch2 < nfull)
                def _(b=b, ch2=ch2):
                    @pl.when(g > 0)
                    def _():
                        drain(b, ch2 - _WINDOW)

                    for e16 in range(8):
                        def store(r, x, b=b, e16=e16):
                            vals[r, b, pl.ds(e16 * 16, 16)] = x
                        build16(ch2 * 128 + e16 * 16, store)
                    fire(b, ch2)

            return carry

        lax.fori_loop(0, ngrp, group_body, 0)

        @pl.when(n_keep > 11904)
        def _():
            for e16 in range(6):
                def store(r, x, e16=e16):
                    vtail[r, pl.ds(e16 * 16, 16)] = x
                build16(11904 + e16 * 16, store)
            for r in range(3):
                pltpu.async_copy(
                    vtail.at[r],
                    o_ref.at[pl.ds(base_out + r * N_PEAKS + 11904, 96)],
                    sem_sc)
            for r in range(3):
                pltpu.make_async_copy(
                    vtail.at[r],
                    o_ref.at[pl.ds(base_out + r * N_PEAKS + 11904, 96)],
                    sem_sc).wait()

        for b in range(_WINDOW):
            @pl.when(b < nfull)
            def _(b=b):
                last_ch = b + _WINDOW * ((nfull - 1 - b) // _WINDOW)
                drain(b, last_ch)


def _compact(p_flat):
    mesh = plsc.VectorSubcoreMesh(core_axis_name="c", subcore_axis_name="s")
    f = functools.partial(
        pl.kernel,
        out_type=jax.ShapeDtypeStruct((_OUT_LEN,), jnp.float32),
        mesh=mesh,
        scratch_types=[
            pltpu.VMEM((_CH,), jnp.float32),
            pltpu.VMEM((_CH,), jnp.float32),
            pltpu.VMEM((_NBUF,), jnp.float32),
            pltpu.VMEM((_NBUF,), jnp.int32),
            pltpu.VMEM((_NBUF,), jnp.float32),
            pltpu.VMEM((_NBUF,), jnp.int32),
            pltpu.VMEM((3, _WINDOW, 128), jnp.float32),
            pltpu.VMEM((3, 96), jnp.float32),
            pltpu.VMEM((2000,), jnp.float32),
            pltpu.VMEM_SHARED((8, _NBUF), jnp.float32),
            pltpu.VMEM_SHARED((8, _NBUF), jnp.int32),
            pltpu.SMEM((1,), jnp.int32),
            pltpu.SemaphoreType.DMA,
            pltpu.SemaphoreType.DMA,
            pltpu.SemaphoreType.DMA,
        ],
        compiler_params=pltpu.CompilerParams(needs_layout_passes=False),
    )(_sc_body)
    return f(p_flat)


def kernel(spec_tensor):
    B, F, T = spec_tensor.shape
    p = _peak_map(spec_tensor)
    out = _compact(jnp.reshape(p, (-1,)))
    return jnp.reshape(out, (B, 3, N_PEAKS))
